# Initial kernel scaffold; baseline (speedup 1.0000x reference)
import functools

import jax
import jax.numpy as jnp
from jax import lax
from jax.experimental import pallas as pl
from jax.experimental.pallas import tpu as pltpu
from jax.experimental.pallas import tpu_sc as plsc

VOCAB = 1000000
HIDDEN = 32
BATCH = 4096
HIST = 200

NC = 2
NS = 16
NW = NC * NS
LANES = 16

TOTAL = BATCH * HIST
IDX_W = 128
GROUPS = TOTAL // IDX_W
GROUPS_PER_W = GROUPS // NW
CHUNK_G = 8
CHUNK_ROWS = CHUNK_G * IDX_W
STEPS = GROUPS_PER_W // CHUNK_G


def _tanh16(v):
    e = jnp.exp(v + v)
    return 1.0 - 2.0 / (e + 1.0)


def _sc_body(sents_hbm, table_hbm, out_hbm, idx_v, rows_v, sem):
    wid = lax.axis_index("s") * NC + lax.axis_index("c")

    def step(c, carry):
        g0 = wid * GROUPS_PER_W + c * CHUNK_G
        pltpu.sync_copy(sents_hbm.at[pl.ds(g0, CHUNK_G)], idx_v)
        descs = [
            pltpu.async_copy(
                table_hbm.at[idx_v.at[j]],
                rows_v.at[pl.ds(j * IDX_W, IDX_W)],
                sem,
            )
            for j in range(CHUNK_G)
        ]
        for d in descs:
            d.wait()

        def compute(r, inner):
            base = r * 8
            for u in range(8):
                row = base + u
                for h in range(HIDDEN // LANES):
                    sl = (row, pl.ds(h * LANES, LANES))
                    rows_v[sl] = _tanh16(rows_v[sl])
            return inner

        lax.fori_loop(0, CHUNK_ROWS // 8, compute, 0)
        pltpu.sync_copy(rows_v, out_hbm.at[pl.ds(g0 * IDX_W, CHUNK_ROWS)])
        return carry

    lax.fori_loop(0, STEPS, step, 0)


@jax.jit
def kernel(sents, table):
    idx = sents.astype(jnp.int32).reshape(GROUPS, IDX_W)
    mesh = plsc.VectorSubcoreMesh(core_axis_name="c", subcore_axis_name="s")
    out = pl.kernel(
        _sc_body,
        out_type=jax.ShapeDtypeStruct((TOTAL, HIDDEN), jnp.float32),
        mesh=mesh,
        scratch_types=[
            pltpu.VMEM((CHUNK_G, IDX_W), jnp.int32),
            pltpu.VMEM((CHUNK_ROWS, HIDDEN), jnp.float32),
            pltpu.SemaphoreType.DMA,
        ],
    )(idx, table)
    return out.reshape(BATCH, HIST, HIDDEN)


# speedup vs baseline: 1.3303x; 1.3303x over previous
import functools

import jax
import jax.numpy as jnp
from jax import lax
from jax.experimental import pallas as pl
from jax.experimental.pallas import tpu as pltpu
from jax.experimental.pallas import tpu_sc as plsc

VOCAB = 1000000
HIDDEN = 32
BATCH = 4096
HIST = 200

NC = 2
NS = 16
NW = NC * NS
LANES = 16

TOTAL = BATCH * HIST
IDX_W = 128
GROUPS = TOTAL // IDX_W
GROUPS_PER_W = GROUPS // NW
CHUNK_G = 8
CHUNK_ROWS = CHUNK_G * IDX_W
STEPS = GROUPS_PER_W // CHUNK_G


def _tanh16(v):
    e = jnp.exp(v + v)
    return 1.0 - 2.0 / (e + 1.0)


def _sc_body(sents_hbm, table_hbm, out_hbm, idx_v, rows_v, sem):
    wid = lax.axis_index("s") * NC + lax.axis_index("c")

    def step(c, carry):
        g0 = wid * GROUPS_PER_W + c * CHUNK_G
        pltpu.sync_copy(sents_hbm.at[pl.ds(g0, CHUNK_G)], idx_v)
        descs = [
            pltpu.async_copy(
                table_hbm.at[idx_v.at[j]],
                rows_v.at[pl.ds(j * IDX_W, IDX_W)],
                sem,
            )
            for j in range(CHUNK_G)
        ]
        for d in descs:
            d.wait()

        def compute(r, inner):
            base = r * 8
            for u in range(8):
                row = base + u
                for h in range(HIDDEN // LANES):
                    sl = (row, pl.ds(h * LANES, LANES))
                    rows_v[sl] = _tanh16(rows_v[sl])
            return inner

        lax.fori_loop(0, CHUNK_ROWS // 8, compute, 0)
        pltpu.sync_copy(rows_v, out_hbm.at[pl.ds(g0 * IDX_W, CHUNK_ROWS)])
        return carry

    lax.fori_loop(0, STEPS, step, 0)


@jax.jit
def kernel(sents, table):
    idx = sents.astype(jnp.int32).reshape(GROUPS, IDX_W)
    mesh = plsc.VectorSubcoreMesh(core_axis_name="c", subcore_axis_name="s")
    out = pl.kernel(
        _sc_body,
        out_type=jax.ShapeDtypeStruct((TOTAL, HIDDEN), jnp.float32),
        mesh=mesh,
        scratch_types=[
            pltpu.VMEM((CHUNK_G, IDX_W), jnp.int32),
            pltpu.VMEM((CHUNK_ROWS, HIDDEN), jnp.float32),
            pltpu.SemaphoreType.DMA,
        ],
        compiler_params=pltpu.CompilerParams(use_tc_tiling_on_sc=False),
    )(idx, table)
    return out.reshape(BATCH, HIST, HIDDEN)


# speedup vs baseline: 1.4018x; 1.0537x over previous
import jax
import jax.numpy as jnp
from jax import lax
from jax.experimental import pallas as pl
from jax.experimental.pallas import tpu as pltpu
from jax.experimental.pallas import tpu_sc as plsc

VOCAB1 = 1000001
HIDDEN = 32
BATCH = 4096
HIST = 200

VB = 2048
NB = (VOCAB1 + VB - 1) // VB
VPAD = NB * VB
P_ROWS = VPAD * HIDDEN // 128

NW = 32
IDX_W = 128
ITEMS = HIST * BATCH // IDX_W
ITEMS_PER_W = ITEMS // NW
CHUNK = 8
STEPS = ITEMS_PER_W // CHUNK
OUT_ROWS = HIST * HIDDEN * BATCH // 128


def _a_body(x_ref, o_ref):
    x = jnp.tanh(x_ref[...])
    xt = jnp.transpose(x, (1, 0))
    xt4 = xt.reshape(VB // 4, 4, 32)
    o_ref[...] = jnp.concatenate([xt4[:, e, :] for e in range(4)], axis=1)


def _relayout_tanh(table_t):
    return pl.pallas_call(
        _a_body,
        grid=(NB,),
        in_specs=[pl.BlockSpec((HIDDEN, VB), lambda i: (0, i))],
        out_specs=pl.BlockSpec((VB * HIDDEN // 128, 128), lambda i: (i, 0)),
        out_shape=jax.ShapeDtypeStruct((P_ROWS, 128), jnp.float32),
    )(table_t)


def _b_body(idx_hbm, tab_hbm, out_hbm, idx_v, x8, t8, gsem, osem):
    wid = lax.axis_index("s") * 2 + lax.axis_index("c")
    iota = lax.iota(jnp.int32, 16)
    row_lo = iota
    row_hi = iota + 16

    def chunk(c, carry):
        r0 = wid * ITEMS_PER_W + c * CHUNK
        pltpu.sync_copy(idx_hbm.at[pl.ds(r0, CHUNK)], idx_v)
        for k in range(CHUNK):
            pltpu.async_copy(tab_hbm.at[idx_v.at[k]], x8.at[k], gsem.at[k])
        @pl.when(c > 0)
        def _():
            for k in range(CHUNK):
                for t in range(4):
                    pltpu.make_async_copy(
                        t8.at[k, pl.ds(t * 8, 8)],
                        out_hbm.at[pl.ds(0, 8)],
                        osem,
                    ).wait()

        for k in range(CHUNK):
            r = r0 + k
            h = (r // 256) * 8 + (r % 8)
            j = (r % 256) // 8
            kidx = jnp.full((16,), k, jnp.int32)
            pltpu.make_async_copy(tab_hbm.at[idx_v.at[k]], x8.at[k], gsem.at[k]).wait()

            def tr(rr, inner):
                for u in range(8):
                    l = rr * 8 + u
                    lidx = jnp.full((16,), l, jnp.int32)
                    plsc.store_scatter(
                        t8, [kidx, row_lo, lidx], x8[k, l, pl.ds(0, 16)]
                    )
                    plsc.store_scatter(
                        t8, [kidx, row_hi, lidx], x8[k, l, pl.ds(16, 16)]
                    )
                return inner

            lax.fori_loop(0, IDX_W // 8, tr, 0)
            obase = h * 1024 + j * 8
            for t in range(4):
                pltpu.async_copy(
                    t8.at[k, pl.ds(t * 8, 8)],
                    out_hbm.at[pl.ds(obase + t * 256, 8)],
                    osem,
                )
        return carry

    lax.fori_loop(0, STEPS, chunk, 0)
    for k in range(CHUNK):
        for t in range(4):
            pltpu.make_async_copy(
                t8.at[k, pl.ds(t * 8, 8)], out_hbm.at[pl.ds(0, 8)], osem
            ).wait()


def _gather_transpose(idx2d, ptab):
    mesh = plsc.VectorSubcoreMesh(core_axis_name="c", subcore_axis_name="s")
    return pl.kernel(
        _b_body,
        out_type=jax.ShapeDtypeStruct((OUT_ROWS, 128), jnp.float32),
        mesh=mesh,
        scratch_types=[
            pltpu.VMEM((CHUNK, IDX_W), jnp.int32),
            pltpu.VMEM((CHUNK, IDX_W, HIDDEN), jnp.float32),
            pltpu.VMEM((CHUNK, HIDDEN, IDX_W), jnp.float32),
            pltpu.SemaphoreType.DMA((CHUNK,)),
            pltpu.SemaphoreType.DMA,
        ],
        compiler_params=pltpu.CompilerParams(
            use_tc_tiling_on_sc=False, needs_layout_passes=False
        ),
    )(idx2d, ptab)


@jax.jit
def kernel(sents, table):
    ptab = _relayout_tanh(jnp.transpose(table))
    ptab_rows = ptab.reshape(VPAD, HIDDEN)
    st = jnp.transpose(sents).astype(jnp.int32)
    idx2d = (
        st.reshape(25, 8, 32, IDX_W)
        .transpose(0, 2, 1, 3)
        .reshape(ITEMS, IDX_W)
    )
    ov = _gather_transpose(idx2d, ptab_rows)
    o6 = ov.reshape(HIST, 4, 32, 8, IDX_W)
    out = o6.transpose(2, 4, 0, 1, 3).reshape(BATCH, HIST, HIDDEN)
    return out


# speedup vs baseline: 1.4054x; 1.0026x over previous
import jax
import jax.numpy as jnp
from jax import lax
from jax.experimental import pallas as pl
from jax.experimental.pallas import tpu as pltpu
from jax.experimental.pallas import tpu_sc as plsc

VOCAB1 = 1000001
HIDDEN = 32
BATCH = 4096
HIST = 200

VB = 2048
NB = (VOCAB1 + VB - 1) // VB
VPAD = NB * VB
P_ROWS = VPAD * HIDDEN // 128

NW = 32
IDX_W = 128
ITEMS = HIST * BATCH // IDX_W
ITEMS_PER_W = ITEMS // NW
CHUNK = 8
STEPS = ITEMS_PER_W // CHUNK
OUT_ROWS = HIST * HIDDEN * BATCH // 128


def _a_body(x_ref, o_ref):
    x = jnp.tanh(x_ref[...])
    xt = jnp.transpose(x, (1, 0))
    xt4 = xt.reshape(VB // 4, 4, 32)
    o_ref[...] = jnp.concatenate([xt4[:, e, :] for e in range(4)], axis=1)


def _relayout_tanh(table_t):
    return pl.pallas_call(
        _a_body,
        grid=(NB,),
        in_specs=[pl.BlockSpec((HIDDEN, VB), lambda i: (0, i))],
        out_specs=pl.BlockSpec((VB * HIDDEN // 128, 128), lambda i: (i, 0)),
        out_shape=jax.ShapeDtypeStruct((P_ROWS, 128), jnp.float32),
    )(table_t)


def _b_body(idx_hbm, tab_hbm, out_hbm, idx_v, x8, t8f, gsem, osem):
    wid = lax.axis_index("s") * 2 + lax.axis_index("c")
    iota128 = lax.iota(jnp.int32, 16) * 128

    def chunk(c, carry):
        r0 = wid * ITEMS_PER_W + c * CHUNK
        pltpu.sync_copy(idx_hbm.at[pl.ds(r0, CHUNK)], idx_v)
        for k in range(CHUNK):
            pltpu.async_copy(tab_hbm.at[idx_v.at[k]], x8.at[k], gsem.at[k])
        @pl.when(c > 0)
        def _():
            for _k in range(CHUNK * 4):
                pltpu.make_async_copy(
                    t8f.at[pl.ds(0, 1024)], out_hbm.at[pl.ds(0, 1024)], osem
                ).wait()

        for k in range(CHUNK):
            r = r0 + k
            h = (r // 256) * 8 + (r % 8)
            j = (r % 256) // 8
            pltpu.make_async_copy(tab_hbm.at[idx_v.at[k]], x8.at[k], gsem.at[k]).wait()

            def tr(rr, a_lo):
                for u in range(8):
                    l = rr * 8 + u
                    plsc.store_scatter(t8f, [a_lo], x8[k, l, pl.ds(0, 16)])
                    plsc.store_scatter(
                        t8f, [a_lo + 2048], x8[k, l, pl.ds(16, 16)]
                    )
                    a_lo = a_lo + 1
                return a_lo

            lax.fori_loop(0, IDX_W // 8, tr, iota128 + (k * 4096))
            obase = (h * 1024 + j * 8) * 128
            for t in range(4):
                pltpu.async_copy(
                    t8f.at[pl.ds(k * 4096 + t * 1024, 1024)],
                    out_hbm.at[pl.ds(obase + t * 32768, 1024)],
                    osem,
                )
        return carry

    lax.fori_loop(0, STEPS, chunk, 0)
    for _k in range(CHUNK * 4):
        pltpu.make_async_copy(
            t8f.at[pl.ds(0, 1024)], out_hbm.at[pl.ds(0, 1024)], osem
        ).wait()


def _gather_transpose(idx2d, ptab):
    mesh = plsc.VectorSubcoreMesh(core_axis_name="c", subcore_axis_name="s")
    return pl.kernel(
        _b_body,
        out_type=jax.ShapeDtypeStruct((OUT_ROWS * 128,), jnp.float32),
        mesh=mesh,
        scratch_types=[
            pltpu.VMEM((CHUNK, IDX_W), jnp.int32),
            pltpu.VMEM((CHUNK, IDX_W, HIDDEN), jnp.float32),
            pltpu.VMEM((CHUNK * HIDDEN * IDX_W,), jnp.float32),
            pltpu.SemaphoreType.DMA((CHUNK,)),
            pltpu.SemaphoreType.DMA,
        ],
        compiler_params=pltpu.CompilerParams(
            use_tc_tiling_on_sc=False, needs_layout_passes=False
        ),
    )(idx2d, ptab)


@jax.jit
def kernel(sents, table):
    ptab = _relayout_tanh(jnp.transpose(table))
    ptab_rows = ptab.reshape(VPAD, HIDDEN)
    st = jnp.transpose(sents).astype(jnp.int32)
    idx2d = (
        st.reshape(25, 8, 32, IDX_W)
        .transpose(0, 2, 1, 3)
        .reshape(ITEMS, IDX_W)
    )
    ov = _gather_transpose(idx2d, ptab_rows)
    o6 = ov.reshape(HIST, 4, 32, 8, IDX_W)
    out = o6.transpose(2, 4, 0, 1, 3).reshape(BATCH, HIST, HIDDEN)
    return out


# speedup vs baseline: 1.9617x; 1.3958x over previous
import jax
import jax.numpy as jnp
from jax import lax
from jax.experimental import pallas as pl
from jax.experimental.pallas import tpu as pltpu
from jax.experimental.pallas import tpu_sc as plsc

VOCAB1 = 1000001
HIDDEN = 32
BATCH = 4096
HIST = 200

VB = 2048
NB = (VOCAB1 + VB - 1) // VB
VPAD = NB * VB
P_ROWS = VPAD * HIDDEN // 128

NW = 32
IDX_W = 128
ITEMS = HIST * BATCH // IDX_W
ITEMS_PER_W = ITEMS // NW
CHUNK = 8
STEPS = ITEMS_PER_W // CHUNK
OUT_ROWS = HIST * HIDDEN * BATCH // 128


def _a_body(x_ref, o_ref):
    x = jnp.tanh(x_ref[...])
    xt = jnp.transpose(x, (1, 0))
    xt4 = xt.reshape(VB // 4, 4, 32)
    o_ref[...] = jnp.concatenate([xt4[:, e, :] for e in range(4)], axis=1)


def _relayout_tanh(table_t):
    return pl.pallas_call(
        _a_body,
        grid=(NB,),
        in_specs=[pl.BlockSpec((HIDDEN, VB), lambda i: (0, i))],
        out_specs=pl.BlockSpec((VB * HIDDEN // 128, 128), lambda i: (i, 0)),
        out_shape=jax.ShapeDtypeStruct((P_ROWS, 128), jnp.float32),
    )(table_t)


def _b_body(idx_hbm, tab_hbm, out_hbm, idx_v, x8, t8f, gsem, osem):
    wid = lax.axis_index("s") * 2 + lax.axis_index("c")
    iota136 = lax.iota(jnp.int32, 16) * 136

    def chunk(c, carry):
        r0 = wid * ITEMS_PER_W + c * CHUNK
        pltpu.sync_copy(idx_hbm.at[pl.ds(r0, CHUNK)], idx_v)
        for k in range(CHUNK):
            pltpu.async_copy(tab_hbm.at[idx_v.at[k]], x8.at[k], gsem.at[k])
        @pl.when(c > 0)
        def _():
            for _k in range(CHUNK):
                pltpu.make_async_copy(
                    t8f.at[pl.ds(0, 4096)], out_hbm.at[pl.ds(0, 4096)], osem
                ).wait()

        for k in range(CHUNK):
            r = r0 + k
            h = (r // 256) * 8 + (r % 8)
            j = (r % 256) // 8
            pltpu.make_async_copy(tab_hbm.at[idx_v.at[k]], x8.at[k], gsem.at[k]).wait()

            def tr(rr, a_lo):
                for u in range(8):
                    l = rr * 8 + u
                    plsc.store_scatter(t8f, [a_lo], x8[k, l, pl.ds(0, 16)])
                    plsc.store_scatter(
                        t8f, [a_lo + 2176], x8[k, l, pl.ds(16, 16)]
                    )
                    a_lo = a_lo + 1
                return a_lo

            lax.fori_loop(0, IDX_W // 8, tr, iota136 + (k * 4352))
            obase = (h * 1024 + j * 8) * 128
            for t in range(4):
                for s in range(8):
                    pltpu.async_copy(
                        t8f.at[pl.ds(k * 4352 + (t * 8 + s) * 136, IDX_W)],
                        out_hbm.at[pl.ds(obase + t * 32768 + s * 128, IDX_W)],
                        osem,
                    )
        return carry

    lax.fori_loop(0, STEPS, chunk, 0)
    for _k in range(CHUNK):
        pltpu.make_async_copy(
            t8f.at[pl.ds(0, 4096)], out_hbm.at[pl.ds(0, 4096)], osem
        ).wait()


def _gather_transpose(idx2d, ptab):
    mesh = plsc.VectorSubcoreMesh(core_axis_name="c", subcore_axis_name="s")
    return pl.kernel(
        _b_body,
        out_type=jax.ShapeDtypeStruct((OUT_ROWS * 128,), jnp.float32),
        mesh=mesh,
        scratch_types=[
            pltpu.VMEM((CHUNK, IDX_W), jnp.int32),
            pltpu.VMEM((CHUNK, IDX_W, HIDDEN), jnp.float32),
            pltpu.VMEM((CHUNK * HIDDEN * 136,), jnp.float32),
            pltpu.SemaphoreType.DMA((CHUNK,)),
            pltpu.SemaphoreType.DMA,
        ],
        compiler_params=pltpu.CompilerParams(
            use_tc_tiling_on_sc=False, needs_layout_passes=False
        ),
    )(idx2d, ptab)


@jax.jit
def kernel(sents, table):
    ptab = _relayout_tanh(jnp.transpose(table))
    ptab_rows = ptab.reshape(VPAD, HIDDEN)
    st = jnp.transpose(sents).astype(jnp.int32)
    idx2d = (
        st.reshape(25, 8, 32, IDX_W)
        .transpose(0, 2, 1, 3)
        .reshape(ITEMS, IDX_W)
    )
    ov = _gather_transpose(idx2d, ptab_rows)
    o6 = ov.reshape(HIST, 4, 32, 8, IDX_W)
    out = o6.transpose(2, 4, 0, 1, 3).reshape(BATCH, HIST, HIDDEN)
    return out


# speedup vs baseline: 2.0101x; 1.0247x over previous
import jax
import jax.numpy as jnp
from jax import lax
from jax.experimental import pallas as pl
from jax.experimental.pallas import tpu as pltpu
from jax.experimental.pallas import tpu_sc as plsc

VOCAB1 = 1000001
HIDDEN = 32
BATCH = 4096
HIST = 200

VB = 2048
NB = (VOCAB1 + VB - 1) // VB
VPAD = NB * VB
P_ROWS = VPAD * HIDDEN // 128

NW = 32
IDX_W = 128
ITEMS = HIST * BATCH // IDX_W
ITEMS_PER_W = ITEMS // NW
CHUNK = 10
STEPS = ITEMS_PER_W // CHUNK
OUT_ROWS = HIST * HIDDEN * BATCH // 128


def _a_body(x_ref, o_ref):
    x = jnp.tanh(x_ref[...])
    xt = jnp.transpose(x, (1, 0))
    xt4 = xt.reshape(VB // 4, 4, 32)
    o_ref[...] = jnp.concatenate([xt4[:, e, :] for e in range(4)], axis=1)


def _relayout_tanh(table_t):
    return pl.pallas_call(
        _a_body,
        grid=(NB,),
        in_specs=[pl.BlockSpec((HIDDEN, VB), lambda i: (0, i))],
        out_specs=pl.BlockSpec((VB * HIDDEN // 128, 128), lambda i: (i, 0)),
        out_shape=jax.ShapeDtypeStruct((P_ROWS, 128), jnp.float32),
    )(table_t)


def _b_body(idx_hbm, tab_hbm, out_hbm, idx_v, x8, t8f, gsem, osem):
    wid = lax.axis_index("s") * 2 + lax.axis_index("c")
    iota136 = lax.iota(jnp.int32, 16) * 136

    def chunk(c, carry):
        r0 = wid * ITEMS_PER_W + c * CHUNK
        pltpu.sync_copy(idx_hbm.at[pl.ds(r0, CHUNK)], idx_v)
        for k in range(CHUNK):
            pltpu.async_copy(tab_hbm.at[idx_v.at[k]], x8.at[k], gsem.at[k])
        @pl.when(c > 0)
        def _():
            for _k in range(CHUNK):
                pltpu.make_async_copy(
                    t8f.at[pl.ds(0, 4096)], out_hbm.at[pl.ds(0, 4096)], osem
                ).wait()

        for k in range(CHUNK):
            r = r0 + k
            h = (r // 256) * 8 + (r % 8)
            j = (r % 256) // 8
            pltpu.make_async_copy(tab_hbm.at[idx_v.at[k]], x8.at[k], gsem.at[k]).wait()

            def tr(rr, a_lo):
                for u in range(8):
                    l = rr * 8 + u
                    plsc.store_scatter(t8f, [a_lo], x8[k, l, pl.ds(0, 16)])
                    plsc.store_scatter(
                        t8f, [a_lo + 2176], x8[k, l, pl.ds(16, 16)]
                    )
                    a_lo = a_lo + 1
                return a_lo

            lax.fori_loop(0, IDX_W // 8, tr, iota136 + (k * 4352))
            obase = (h * 1024 + j * 8) * 128
            for t in range(4):
                for s in range(8):
                    pltpu.async_copy(
                        t8f.at[pl.ds(k * 4352 + (t * 8 + s) * 136, IDX_W)],
                        out_hbm.at[pl.ds(obase + t * 32768 + s * 128, IDX_W)],
                        osem,
                    )
        return carry

    lax.fori_loop(0, STEPS, chunk, 0)
    for _k in range(CHUNK):
        pltpu.make_async_copy(
            t8f.at[pl.ds(0, 4096)], out_hbm.at[pl.ds(0, 4096)], osem
        ).wait()


def _gather_transpose(idx2d, ptab):
    mesh = plsc.VectorSubcoreMesh(core_axis_name="c", subcore_axis_name="s")
    return pl.kernel(
        _b_body,
        out_type=jax.ShapeDtypeStruct((OUT_ROWS * 128,), jnp.float32),
        mesh=mesh,
        scratch_types=[
            pltpu.VMEM((CHUNK, IDX_W), jnp.int32),
            pltpu.VMEM((CHUNK, IDX_W, HIDDEN), jnp.float32),
            pltpu.VMEM((CHUNK * HIDDEN * 136,), jnp.float32),
            pltpu.SemaphoreType.DMA((CHUNK,)),
            pltpu.SemaphoreType.DMA,
        ],
        compiler_params=pltpu.CompilerParams(
            use_tc_tiling_on_sc=False, needs_layout_passes=False
        ),
    )(idx2d, ptab)


@jax.jit
def kernel(sents, table):
    ptab = _relayout_tanh(jnp.transpose(table))
    ptab_rows = ptab.reshape(VPAD, HIDDEN)
    st = jnp.transpose(sents).astype(jnp.int32)
    idx2d = (
        st.reshape(25, 8, 32, IDX_W)
        .transpose(0, 2, 1, 3)
        .reshape(ITEMS, IDX_W)
    )
    ov = _gather_transpose(idx2d, ptab_rows)
    o6 = ov.reshape(HIST, 4, 32, 8, IDX_W)
    out = o6.transpose(2, 4, 0, 1, 3).reshape(BATCH, HIST, HIDDEN)
    return out
